# Initial kernel scaffold; baseline (speedup 1.0000x reference)
import jax
import jax.numpy as jnp
from jax.experimental import pallas as pl


def kernel(x, W):
    raise NotImplementedError("write your pallas kernel here")


# speedup vs baseline: 2.7260x; 2.7260x over previous
import jax
import jax.numpy as jnp
import numpy as np
from jax.experimental import pallas as pl
from jax.experimental.pallas import tpu as pltpu

_B, _S, _D = 4, 8192, 1024
_K = _S // 2
_BS = 2048
_NSTEPS = (_B * _S) // _BS
_ROWS = _B * _S
_IMIN = np.int32(-2147483648)


def _body(x_ref, w_ref, mask_ref, weights_ref, aux_ref, logits_ref, acc_ref):
    i = pl.program_id(0)
    xb = x_ref[...]
    wv = w_ref[...]
    lg = jax.lax.dot_general(
        wv, xb, (((1,), (1,)), ((), ())),
        preferred_element_type=jnp.float32)

    wgt = 1.0 / (1.0 + jnp.exp(-lg))
    weights_ref[...] = wgt[None]

    @pl.when(i == 0)
    def _init():
        acc_ref[0] = 0.0
        acc_ref[1] = 0.0

    v = wgt - 0.5
    acc_ref[0] += jnp.sum(v)
    acc_ref[1] += jnp.sum(v * v)

    b = i // (_S // _BS)
    c = i % (_S // _BS)
    logits_ref[pl.ds(b, 1), pl.ds(c * _BS, _BS)] = lg

    @pl.when(i == _NSTEPS - 1)
    def _select():
        lgs = logits_ref[...]
        ibits = pltpu.bitcast(lgs, jnp.int32)
        skeys = jnp.where(ibits < 0, ibits ^ jnp.int32(0x7FFFFFFF), ibits)

        prefix = jnp.zeros((_B, 1), jnp.int32)
        for bit in range(31, -1, -1):
            cand = prefix | (_IMIN if bit == 31 else np.int32(1 << bit))
            scand = cand ^ _IMIN
            cnt = jnp.sum((skeys >= scand).astype(jnp.int32), axis=1,
                          keepdims=True)
            prefix = jnp.where(cnt >= _K, cand, prefix)
        sT = prefix ^ _IMIN

        cnt_gt = jnp.sum((skeys > sT).astype(jnp.int32), axis=1, keepdims=True)
        need = _K - cnt_gt
        eq = skeys == sT
        idx = jax.lax.broadcasted_iota(jnp.int32, (_B, _S), 1)
        jpref = jnp.zeros((_B, 1), jnp.int32)
        for bit in range(12, -1, -1):
            cand = jpref | jnp.int32(1 << bit)
            c = jnp.sum((eq & (idx < cand)).astype(jnp.int32), axis=1,
                        keepdims=True)
            jpref = jnp.where(c < need, cand, jpref)
        jstar = jpref + 1

        sel = (skeys > sT) | (eq & (idx < jstar))
        mask_ref[...] = sel.astype(jnp.float32)

        n = jnp.float32(_ROWS)
        aux_ref[0, 0] = (acc_ref[1] - acc_ref[0] * acc_ref[0] / n) / (n - 1.0)


def kernel(x, W):
    xf = x.reshape(_ROWS, _D)
    mask2d, w2d, aux = pl.pallas_call(
        _body,
        grid=(_NSTEPS,),
        in_specs=[
            pl.BlockSpec((_BS, _D), lambda i: (i, 0)),
            pl.BlockSpec((1, _D), lambda i: (0, 0)),
        ],
        out_specs=[
            pl.BlockSpec((_B, _S), lambda i: (0, 0)),
            pl.BlockSpec((1, 1, _BS), lambda i: (i, 0, 0)),
            pl.BlockSpec(memory_space=pltpu.SMEM),
        ],
        out_shape=[
            jax.ShapeDtypeStruct((_B, _S), jnp.float32),
            jax.ShapeDtypeStruct((_NSTEPS, 1, _BS), jnp.float32),
            jax.ShapeDtypeStruct((1, 1), jnp.float32),
        ],
        scratch_shapes=[
            pltpu.VMEM((_B, _S), jnp.float32),
            pltpu.SMEM((2,), jnp.float32),
        ],
        compiler_params=pltpu.CompilerParams(
            dimension_semantics=("arbitrary",)),
    )(xf, W)
    mask = mask2d.reshape(_B, _S, 1)
    weights = w2d.reshape(_B, _S, 1)
    aux_loss = aux.reshape(())
    return (mask, weights, aux_loss)
